# Initial kernel scaffold; baseline (speedup 1.0000x reference)
import jax
import jax.numpy as jnp
from jax.experimental import pallas as pl


def kernel(x, edge_index, edge_weight, W0, b0, W1, b1, W2, b2, Wout, bout):
    raise NotImplementedError("write your pallas kernel here")


# speedup vs baseline: 22.3039x; 22.3039x over previous
import functools

import jax
import jax.numpy as jnp
from jax import lax
from jax.experimental import pallas as pl
from jax.experimental.pallas import tpu as pltpu
from jax.experimental.pallas import tpu_sc as plsc

N = 100000
E = 1600000
F_IN = 128
H = 32
C = 2

NC = 2
NS = 16
L = 16

IB = 128
CH = 1024
E_PAD = 1605632
NBTOT = E_PAD // IB
DEG_ROWS_W = NBTOT // (NC * NS)
AGG_ROWS_S = NBTOT // NS

_mesh = plsc.VectorSubcoreMesh(core_axis_name="c", subcore_axis_name="s")

NP = 100096
_TILE_FULL = 6400
_TILE_LAST = 4096
_ZROWS = 256


@functools.partial(
    pl.kernel,
    out_type=jax.ShapeDtypeStruct((NC, NP), jnp.float32),
    mesh=_mesh,
    scratch_types=[
        pltpu.VMEM((8, IB), jnp.int32),
        pltpu.VMEM((CH,), jnp.float32),
        pltpu.VMEM((_ZROWS,), jnp.float32),
        pltpu.VMEM_SHARED((NP,), jnp.float32),
    ],
    compiler_params=pltpu.CompilerParams(use_tc_tiling_on_sc=False),
)
def _deg_kernel(dst_hbm, ew_hbm, out_hbm, idx_v, val_v, zero_v, dacc):
    cid = lax.axis_index("c")
    sid = lax.axis_index("s")
    wid = sid * NC + cid

    def zfill(i, _):
        zero_v[pl.ds(i * L, L)] = jnp.zeros((L,), jnp.float32)
        return 0
    lax.fori_loop(0, _ZROWS // L, zfill, 0)

    @pl.when(sid < NS - 1)
    def _():
        for r in range(_TILE_FULL // _ZROWS):
            pltpu.sync_copy(zero_v, dacc.at[pl.ds(sid * _TILE_FULL + r * _ZROWS, _ZROWS)])

    @pl.when(sid == NS - 1)
    def _():
        for r in range(_TILE_LAST // _ZROWS):
            pltpu.sync_copy(zero_v, dacc.at[pl.ds((NS - 1) * _TILE_FULL + r * _ZROWS, _ZROWS)])

    plsc.subcore_barrier()

    def chunk(g, _):
        row0 = wid * DEG_ROWS_W + g * 8
        pltpu.sync_copy(dst_hbm.at[pl.ds(row0, 8)], idx_v)
        pltpu.sync_copy(ew_hbm.at[pl.ds(row0 * IB, CH)], val_v)
        for j in range(8):
            pltpu.sync_copy(val_v.at[pl.ds(j * IB, IB)], dacc.at[idx_v.at[j]], add=True)
        return 0
    lax.fori_loop(0, DEG_ROWS_W // 8, chunk, 0)

    plsc.subcore_barrier()

    @pl.when(sid < NS - 1)
    def _():
        pltpu.sync_copy(dacc.at[pl.ds(sid * _TILE_FULL, _TILE_FULL)],
                        out_hbm.at[cid, pl.ds(sid * _TILE_FULL, _TILE_FULL)])

    @pl.when(sid == NS - 1)
    def _():
        pltpu.sync_copy(dacc.at[pl.ds((NS - 1) * _TILE_FULL, _TILE_LAST)],
                        out_hbm.at[cid, pl.ds((NS - 1) * _TILE_FULL, _TILE_LAST)])


@functools.partial(
    pl.kernel,
    out_type=jax.ShapeDtypeStruct((NC, NP, L), jnp.float32),
    mesh=_mesh,
    scratch_types=[
        pltpu.VMEM((8, IB), jnp.int32),
        pltpu.VMEM((8, IB), jnp.int32),
        pltpu.VMEM((CH,), jnp.float32),
        pltpu.VMEM((CH, L), jnp.float32),
        pltpu.VMEM((_ZROWS, L), jnp.float32),
        pltpu.VMEM_SHARED((NP, L), jnp.float32),
        pltpu.SemaphoreType.DMA,
    ],
    compiler_params=pltpu.CompilerParams(use_tc_tiling_on_sc=False),
)
def _agg_kernel(src_hbm, dst_hbm, ew_hbm, y_hbm, out_hbm,
                sidx_v, didx_v, ew_v, rows_v, zero_v, acc, sem):
    cid = lax.axis_index("c")
    sid = lax.axis_index("s")

    def zfill(i, _):
        zero_v[i] = jnp.zeros((L,), jnp.float32)
        return 0
    lax.fori_loop(0, _ZROWS, zfill, 0)

    @pl.when(sid < NS - 1)
    def _():
        for r in range(_TILE_FULL // _ZROWS):
            pltpu.sync_copy(zero_v, acc.at[pl.ds(sid * _TILE_FULL + r * _ZROWS, _ZROWS)])

    @pl.when(sid == NS - 1)
    def _():
        for r in range(_TILE_LAST // _ZROWS):
            pltpu.sync_copy(zero_v, acc.at[pl.ds((NS - 1) * _TILE_FULL + r * _ZROWS, _ZROWS)])

    plsc.subcore_barrier()

    yoff = cid * N

    def chunk(g, _):
        row0 = sid * AGG_ROWS_S + g * 8
        pltpu.sync_copy(src_hbm.at[pl.ds(row0, 8)], sidx_v)
        pltpu.sync_copy(dst_hbm.at[pl.ds(row0, 8)], didx_v)
        pltpu.sync_copy(ew_hbm.at[pl.ds(row0 * IB, CH)], ew_v)
        for j in range(8):
            def adj(k, _):
                sidx_v[j, pl.ds(k * L, L)] = sidx_v[j, pl.ds(k * L, L)] + yoff
                return 0
            lax.fori_loop(0, IB // L, adj, 0)
        descs = [
            pltpu.async_copy(y_hbm.at[sidx_v.at[j]],
                             rows_v.at[pl.ds(j * IB, IB)], sem)
            for j in range(8)
        ]
        for d in descs:
            d.wait()

        def scale(gg, _):
            ewv = ew_v[pl.ds(gg * L, L)]
            for l in range(L):
                rows_v[gg * L + l] = rows_v[gg * L + l] * ewv[l]
            return 0
        lax.fori_loop(0, CH // L, scale, 0)

        for j in range(8):
            pltpu.sync_copy(rows_v.at[pl.ds(j * IB, IB)],
                            acc.at[didx_v.at[j]], add=True)
        return 0
    lax.fori_loop(0, AGG_ROWS_S // 8, chunk, 0)

    plsc.subcore_barrier()

    @pl.when(sid < NS - 1)
    def _():
        pltpu.sync_copy(acc.at[pl.ds(sid * _TILE_FULL, _TILE_FULL)],
                        out_hbm.at[cid, pl.ds(sid * _TILE_FULL, _TILE_FULL)])

    @pl.when(sid == NS - 1)
    def _():
        pltpu.sync_copy(acc.at[pl.ds((NS - 1) * _TILE_FULL, _TILE_LAST)],
                        out_hbm.at[cid, pl.ds((NS - 1) * _TILE_FULL, _TILE_LAST)])


BN = 2048
_GRID = (N + BN - 1) // BN


def _dis_from(dp_ref):
    deg = dp_ref[0, :] + dp_ref[1, :] + 1.0
    return lax.rsqrt(deg)


def _pre_body(x_ref, w0_ref, b0_ref, w1_ref, dp_ref, h0_ref, xw1_ref, y_ref):
    h0 = jnp.maximum(jnp.dot(x_ref[...], w0_ref[...],
                             preferred_element_type=jnp.float32) + b0_ref[...], 0.0)
    h0_ref[...] = h0
    xw1 = jnp.dot(h0, w1_ref[...], preferred_element_type=jnp.float32)
    xw1_ref[...] = xw1
    dis = _dis_from(dp_ref)
    y = xw1 * dis[:, None]
    y_ref[0] = y[:, :L]
    y_ref[1] = y[:, L:]


def _mid_body(acc_ref, xw1_ref, dp_ref, b1_ref, w2_ref, h1_ref, xw2_ref, y2_ref):
    dis = _dis_from(dp_ref)
    acc = jnp.concatenate([acc_ref[0], acc_ref[1]], axis=1)
    xw1 = xw1_ref[...]
    h1 = acc * dis[:, None] + xw1 * (dis * dis)[:, None] + b1_ref[...]
    h1_ref[...] = h1
    xw2 = jnp.dot(h1, w2_ref[...], preferred_element_type=jnp.float32)
    xw2_ref[...] = xw2
    y2 = xw2 * dis[:, None]
    y2_ref[0] = y2[:, :L]
    y2_ref[1] = y2[:, L:]


def _post_body(h0_ref, h1_ref, acc_ref, xw2_ref, dp_ref, b2_ref, wout_ref,
               bout_ref, out_ref):
    dis = _dis_from(dp_ref)
    acc = jnp.concatenate([acc_ref[0], acc_ref[1]], axis=1)
    xw2 = xw2_ref[...]
    h2 = acc * dis[:, None] + xw2 * (dis * dis)[:, None] + b2_ref[...]
    xx = jnp.concatenate([h0_ref[...], h1_ref[...], h2], axis=1)
    logits = jnp.dot(xx, wout_ref[...],
                     preferred_element_type=jnp.float32) + bout_ref[...]
    m = jnp.max(logits, axis=-1, keepdims=True)
    z = logits - m
    out_ref[...] = z - jnp.log(jnp.sum(jnp.exp(z), axis=-1, keepdims=True))


def _row_spec(w):
    return pl.BlockSpec((BN, w), lambda i: (i, 0))


def _full_spec(shape):
    return pl.BlockSpec(shape, lambda i: tuple(0 for _ in shape))


_dp_spec = pl.BlockSpec((NC, BN), lambda i: (0, i))
_y_spec = pl.BlockSpec((NC, BN, L), lambda i: (0, i, 0))

_pre = pl.pallas_call(
    _pre_body,
    grid=_GRID,
    in_specs=[_row_spec(F_IN), _full_spec((F_IN, H)), _full_spec((1, H)),
              _full_spec((H, H)), _dp_spec],
    out_specs=[_row_spec(H), _row_spec(H), _y_spec],
    out_shape=[jax.ShapeDtypeStruct((N, H), jnp.float32),
               jax.ShapeDtypeStruct((N, H), jnp.float32),
               jax.ShapeDtypeStruct((NC, N, L), jnp.float32)],
)

_mid = pl.pallas_call(
    _mid_body,
    grid=_GRID,
    in_specs=[_y_spec, _row_spec(H), _dp_spec, _full_spec((1, H)),
              _full_spec((H, H))],
    out_specs=[_row_spec(H), _row_spec(H), _y_spec],
    out_shape=[jax.ShapeDtypeStruct((N, H), jnp.float32),
               jax.ShapeDtypeStruct((N, H), jnp.float32),
               jax.ShapeDtypeStruct((NC, N, L), jnp.float32)],
)

_post = pl.pallas_call(
    _post_body,
    grid=_GRID,
    in_specs=[_row_spec(H), _row_spec(H), _y_spec, _row_spec(H), _dp_spec,
              _full_spec((1, H)), _full_spec((3 * H, C)), _full_spec((1, C))],
    out_specs=_row_spec(C),
    out_shape=jax.ShapeDtypeStruct((N, C), jnp.float32),
)


def kernel(x, edge_index, edge_weight, W0, b0, W1, b1, W2, b2, Wout, bout):
    src = edge_index[0].astype(jnp.int32)
    dst = edge_index[1].astype(jnp.int32)
    npad = E_PAD - E
    pad_idx = (jnp.arange(npad, dtype=jnp.int32) * 31) % N
    src_p = jnp.concatenate([src, pad_idx]).reshape(NBTOT, IB)
    dst_p = jnp.concatenate([dst, pad_idx]).reshape(NBTOT, IB)
    ew_p = jnp.concatenate(
        [edge_weight.astype(jnp.float32), jnp.zeros((npad,), jnp.float32)])

    dpart = _deg_kernel(dst_p, ew_p)

    h0, xw1, y1 = _pre(x, W0, b0.reshape(1, H), W1, dpart)
    acc1 = _agg_kernel(src_p, dst_p, ew_p, y1.reshape(NC * N, L))
    h1, xw2, y2 = _mid(acc1, xw1, dpart, b1.reshape(1, H), W2)
    acc2 = _agg_kernel(src_p, dst_p, ew_p, y2.reshape(NC * N, L))
    out = _post(h0, h1, acc2, xw2, dpart, b2.reshape(1, H), Wout,
                bout.reshape(1, C))
    return out


# speedup vs baseline: 29.7815x; 1.3353x over previous
import functools

import jax
import jax.numpy as jnp
from jax import lax
from jax.experimental import pallas as pl
from jax.experimental.pallas import tpu as pltpu
from jax.experimental.pallas import tpu_sc as plsc

N = 100000
E = 1600000
F_IN = 128
H = 32
C = 2

NC = 2
NS = 16
L = 16

IB = 128
CH = 512
E_PAD = 1605632
NBTOT = E_PAD // IB
DEG_ROWS_W = NBTOT // (NC * NS)
AGG_ROWS_S = NBTOT // NS

_mesh = plsc.VectorSubcoreMesh(core_axis_name="c", subcore_axis_name="s")

NP = 100096
_TILE_FULL = 6400
_TILE_LAST = 4096
_ZROWS = 256


DCH = 4
DNCH = DEG_ROWS_W // DCH


@functools.partial(
    pl.kernel,
    out_type=jax.ShapeDtypeStruct((NC, NP), jnp.float32),
    mesh=_mesh,
    scratch_types=[
        pltpu.VMEM((DCH, IB), jnp.int32),
        pltpu.VMEM((DCH, IB), jnp.int32),
        pltpu.VMEM((DCH * IB,), jnp.float32),
        pltpu.VMEM((DCH * IB,), jnp.float32),
        pltpu.VMEM((_ZROWS,), jnp.float32),
        pltpu.VMEM_SHARED((NP,), jnp.float32),
        pltpu.SemaphoreType.DMA,
        pltpu.SemaphoreType.DMA,
        pltpu.SemaphoreType.DMA,
        pltpu.SemaphoreType.DMA,
    ],
    compiler_params=pltpu.CompilerParams(use_tc_tiling_on_sc=False),
)
def _deg_kernel(dst_hbm, ew_hbm, out_hbm, idx0, idx1, val0, val1, zero_v,
                dacc, seml0, seml1, sems0, sems1):
    cid = lax.axis_index("c")
    sid = lax.axis_index("s")
    wid = sid * NC + cid

    def zfill(i, _):
        zero_v[pl.ds(i * L, L)] = jnp.zeros((L,), jnp.float32)
        return 0
    lax.fori_loop(0, _ZROWS // L, zfill, 0)

    @pl.when(sid < NS - 1)
    def _():
        for r in range(_TILE_FULL // _ZROWS):
            pltpu.sync_copy(zero_v, dacc.at[pl.ds(sid * _TILE_FULL + r * _ZROWS, _ZROWS)])

    @pl.when(sid == NS - 1)
    def _():
        for r in range(_TILE_LAST // _ZROWS):
            pltpu.sync_copy(zero_v, dacc.at[pl.ds((NS - 1) * _TILE_FULL + r * _ZROWS, _ZROWS)])

    plsc.subcore_barrier()

    bufs = ((idx0, val0, seml0, sems0), (idx1, val1, seml1, sems1))

    def lin_start(g, bf):
        row0 = wid * DEG_ROWS_W + g * DCH
        pltpu.async_copy(dst_hbm.at[pl.ds(row0, DCH)], bf[0], bf[2])
        pltpu.async_copy(ew_hbm.at[pl.ds(row0 * IB, DCH * IB)], bf[1], bf[2])

    def lin_wait(bf):
        pltpu.make_async_copy(dst_hbm.at[pl.ds(0, DCH)], bf[0], bf[2]).wait()
        pltpu.make_async_copy(ew_hbm.at[pl.ds(0, DCH * IB)], bf[1], bf[2]).wait()

    lin_start(0, bufs[0])

    def outer(g2, _):
        for b in (0, 1):
            g = g2 * 2 + b
            bf = bufs[b]
            bn = bufs[1 - b]
            lin_wait(bf)

            @pl.when(g < DNCH - 1)
            def _():
                lin_start(g + 1, bn)
            sd = [pltpu.async_copy(bf[1].at[pl.ds(j * IB, IB)],
                                   dacc.at[bf[0].at[j]], bf[3], add=True)
                  for j in range(DCH)]
            for d in sd:
                d.wait()
        return 0
    lax.fori_loop(0, DNCH // 2, outer, 0)

    plsc.subcore_barrier()

    @pl.when(sid < NS - 1)
    def _():
        pltpu.sync_copy(dacc.at[pl.ds(sid * _TILE_FULL, _TILE_FULL)],
                        out_hbm.at[cid, pl.ds(sid * _TILE_FULL, _TILE_FULL)])

    @pl.when(sid == NS - 1)
    def _():
        pltpu.sync_copy(dacc.at[pl.ds((NS - 1) * _TILE_FULL, _TILE_LAST)],
                        out_hbm.at[cid, pl.ds((NS - 1) * _TILE_FULL, _TILE_LAST)])


NB_CH = CH // IB
NCH_T = AGG_ROWS_S // NB_CH


@functools.partial(
    pl.kernel,
    out_type=jax.ShapeDtypeStruct((NC, NP, L), jnp.float32),
    mesh=_mesh,
    scratch_types=[
        pltpu.VMEM((NB_CH, IB), jnp.int32),
        pltpu.VMEM((NB_CH, IB), jnp.int32),
        pltpu.VMEM((NB_CH, IB), jnp.int32),
        pltpu.VMEM((NB_CH, IB), jnp.int32),
        pltpu.VMEM((CH,), jnp.float32),
        pltpu.VMEM((CH,), jnp.float32),
        pltpu.VMEM((CH, L), jnp.float32),
        pltpu.VMEM((CH, L), jnp.float32),
        pltpu.VMEM((_ZROWS, L), jnp.float32),
        pltpu.VMEM_SHARED((NP, L), jnp.float32),
        pltpu.SemaphoreType.DMA,
        pltpu.SemaphoreType.DMA,
        pltpu.SemaphoreType.DMA,
        pltpu.SemaphoreType.DMA,
        pltpu.SemaphoreType.DMA,
        pltpu.SemaphoreType.DMA,
    ],
    compiler_params=pltpu.CompilerParams(use_tc_tiling_on_sc=False),
)
def _agg_kernel(src_hbm, dst_hbm, ew_hbm, y_hbm, out_hbm,
                sidx0, sidx1, didx0, didx1, ew0, ew1, rows0, rows1,
                zero_v, acc, seml0, seml1, semg0, semg1, sems0, sems1):
    cid = lax.axis_index("c")
    sid = lax.axis_index("s")

    def zfill(i, _):
        zero_v[i] = jnp.zeros((L,), jnp.float32)
        return 0
    lax.fori_loop(0, _ZROWS, zfill, 0)

    @pl.when(sid < NS - 1)
    def _():
        for r in range(_TILE_FULL // _ZROWS):
            pltpu.sync_copy(zero_v, acc.at[pl.ds(sid * _TILE_FULL + r * _ZROWS, _ZROWS)])

    @pl.when(sid == NS - 1)
    def _():
        for r in range(_TILE_LAST // _ZROWS):
            pltpu.sync_copy(zero_v, acc.at[pl.ds((NS - 1) * _TILE_FULL + r * _ZROWS, _ZROWS)])

    plsc.subcore_barrier()

    yoff = cid * N
    bufs = ((sidx0, didx0, ew0, rows0, seml0, semg0, sems0),
            (sidx1, didx1, ew1, rows1, seml1, semg1, sems1))

    def lin_start(g, bf):
        row0 = sid * AGG_ROWS_S + g * NB_CH
        pltpu.async_copy(src_hbm.at[pl.ds(row0, NB_CH)], bf[0], bf[4])
        pltpu.async_copy(dst_hbm.at[pl.ds(row0, NB_CH)], bf[1], bf[4])
        pltpu.async_copy(ew_hbm.at[pl.ds(row0 * IB, CH)], bf[2], bf[4])

    def lin_wait(bf):
        pltpu.make_async_copy(src_hbm.at[pl.ds(0, NB_CH)], bf[0], bf[4]).wait()
        pltpu.make_async_copy(dst_hbm.at[pl.ds(0, NB_CH)], bf[1], bf[4]).wait()
        pltpu.make_async_copy(ew_hbm.at[pl.ds(0, CH)], bf[2], bf[4]).wait()

    def adjust(bf):
        for j in range(NB_CH):
            def adj(k, _):
                bf[0][j, pl.ds(k * L, L)] = bf[0][j, pl.ds(k * L, L)] + yoff
                return 0
            lax.fori_loop(0, IB // L, adj, 0)

    def gather_start(bf):
        return [pltpu.async_copy(y_hbm.at[bf[0].at[j]],
                                 bf[3].at[pl.ds(j * IB, IB)], bf[5])
                for j in range(NB_CH)]

    def scale_batch(bf, j):
        def scale(gg, _):
            base = j * IB + gg * L
            ewv = bf[2][pl.ds(base, L)]
            for l in range(L):
                bf[3][base + l] = bf[3][base + l] * ewv[l]
            return 0
        lax.fori_loop(0, IB // L, scale, 0)

    def scatter_start(bf, j):
        return pltpu.async_copy(bf[3].at[pl.ds(j * IB, IB)],
                                acc.at[bf[1].at[j]], bf[6], add=True)

    lin_start(0, bufs[0])

    def outer(g2, _):
        for b in (0, 1):
            g = g2 * 2 + b
            bf = bufs[b]
            bn = bufs[1 - b]
            lin_wait(bf)
            adjust(bf)
            gd = gather_start(bf)

            @pl.when(g < NCH_T - 1)
            def _():
                lin_start(g + 1, bn)
            sd = []
            for j in range(NB_CH):
                gd[j].wait()
                scale_batch(bf, j)
                sd.append(scatter_start(bf, j))
            for d in sd:
                d.wait()
        return 0
    lax.fori_loop(0, NCH_T // 2, outer, 0)

    plsc.subcore_barrier()

    @pl.when(sid < NS - 1)
    def _():
        pltpu.sync_copy(acc.at[pl.ds(sid * _TILE_FULL, _TILE_FULL)],
                        out_hbm.at[cid, pl.ds(sid * _TILE_FULL, _TILE_FULL)])

    @pl.when(sid == NS - 1)
    def _():
        pltpu.sync_copy(acc.at[pl.ds((NS - 1) * _TILE_FULL, _TILE_LAST)],
                        out_hbm.at[cid, pl.ds((NS - 1) * _TILE_FULL, _TILE_LAST)])


BN = 2048
_GRID = (N + BN - 1) // BN


def _dis_from(dp_ref):
    deg = dp_ref[0, :] + dp_ref[1, :] + 1.0
    return lax.rsqrt(deg)


def _pre_body(x_ref, w0_ref, b0_ref, w1_ref, dp_ref, h0_ref, xw1_ref, y_ref):
    h0 = jnp.maximum(jnp.dot(x_ref[...], w0_ref[...],
                             preferred_element_type=jnp.float32) + b0_ref[...], 0.0)
    h0_ref[...] = h0
    xw1 = jnp.dot(h0, w1_ref[...], preferred_element_type=jnp.float32)
    xw1_ref[...] = xw1
    dis = _dis_from(dp_ref)
    y = xw1 * dis[:, None]
    y_ref[0] = y[:, :L]
    y_ref[1] = y[:, L:]


def _mid_body(acc_ref, xw1_ref, dp_ref, b1_ref, w2_ref, h1_ref, xw2_ref, y2_ref):
    dis = _dis_from(dp_ref)
    acc = jnp.concatenate([acc_ref[0], acc_ref[1]], axis=1)
    xw1 = xw1_ref[...]
    h1 = acc * dis[:, None] + xw1 * (dis * dis)[:, None] + b1_ref[...]
    h1_ref[...] = h1
    xw2 = jnp.dot(h1, w2_ref[...], preferred_element_type=jnp.float32)
    xw2_ref[...] = xw2
    y2 = xw2 * dis[:, None]
    y2_ref[0] = y2[:, :L]
    y2_ref[1] = y2[:, L:]


def _post_body(h0_ref, h1_ref, acc_ref, xw2_ref, dp_ref, b2_ref, wout_ref,
               bout_ref, out_ref):
    dis = _dis_from(dp_ref)
    acc = jnp.concatenate([acc_ref[0], acc_ref[1]], axis=1)
    xw2 = xw2_ref[...]
    h2 = acc * dis[:, None] + xw2 * (dis * dis)[:, None] + b2_ref[...]
    xx = jnp.concatenate([h0_ref[...], h1_ref[...], h2], axis=1)
    logits = jnp.dot(xx, wout_ref[...],
                     preferred_element_type=jnp.float32) + bout_ref[...]
    m = jnp.max(logits, axis=-1, keepdims=True)
    z = logits - m
    out_ref[...] = z - jnp.log(jnp.sum(jnp.exp(z), axis=-1, keepdims=True))


def _row_spec(w):
    return pl.BlockSpec((BN, w), lambda i: (i, 0))


def _full_spec(shape):
    return pl.BlockSpec(shape, lambda i: tuple(0 for _ in shape))


_dp_spec = pl.BlockSpec((NC, BN), lambda i: (0, i))
_y_spec = pl.BlockSpec((NC, BN, L), lambda i: (0, i, 0))

_pre = pl.pallas_call(
    _pre_body,
    grid=_GRID,
    in_specs=[_row_spec(F_IN), _full_spec((F_IN, H)), _full_spec((1, H)),
              _full_spec((H, H)), _dp_spec],
    out_specs=[_row_spec(H), _row_spec(H), _y_spec],
    out_shape=[jax.ShapeDtypeStruct((N, H), jnp.float32),
               jax.ShapeDtypeStruct((N, H), jnp.float32),
               jax.ShapeDtypeStruct((NC, N, L), jnp.float32)],
)

_mid = pl.pallas_call(
    _mid_body,
    grid=_GRID,
    in_specs=[_y_spec, _row_spec(H), _dp_spec, _full_spec((1, H)),
              _full_spec((H, H))],
    out_specs=[_row_spec(H), _row_spec(H), _y_spec],
    out_shape=[jax.ShapeDtypeStruct((N, H), jnp.float32),
               jax.ShapeDtypeStruct((N, H), jnp.float32),
               jax.ShapeDtypeStruct((NC, N, L), jnp.float32)],
)

_post = pl.pallas_call(
    _post_body,
    grid=_GRID,
    in_specs=[_row_spec(H), _row_spec(H), _y_spec, _row_spec(H), _dp_spec,
              _full_spec((1, H)), _full_spec((3 * H, C)), _full_spec((1, C))],
    out_specs=_row_spec(C),
    out_shape=jax.ShapeDtypeStruct((N, C), jnp.float32),
)


def kernel(x, edge_index, edge_weight, W0, b0, W1, b1, W2, b2, Wout, bout):
    src = edge_index[0].astype(jnp.int32)
    dst = edge_index[1].astype(jnp.int32)
    npad = E_PAD - E
    pad_idx = (jnp.arange(npad, dtype=jnp.int32) * 31) % N
    src_p = jnp.concatenate([src, pad_idx]).reshape(NBTOT, IB)
    dst_p = jnp.concatenate([dst, pad_idx]).reshape(NBTOT, IB)
    ew_p = jnp.concatenate(
        [edge_weight.astype(jnp.float32), jnp.zeros((npad,), jnp.float32)])

    dpart = _deg_kernel(dst_p, ew_p)

    h0, xw1, y1 = _pre(x, W0, b0.reshape(1, H), W1, dpart)
    acc1 = _agg_kernel(src_p, dst_p, ew_p, y1.reshape(NC * N, L))
    h1, xw2, y2 = _mid(acc1, xw1, dpart, b1.reshape(1, H), W2)
    acc2 = _agg_kernel(src_p, dst_p, ew_p, y2.reshape(NC * N, L))
    out = _post(h0, h1, acc2, xw2, dpart, b2.reshape(1, H), Wout,
                bout.reshape(1, C))
    return out


# speedup vs baseline: 35.5077x; 1.1923x over previous
import functools

import jax
import jax.numpy as jnp
from jax import lax
from jax.experimental import pallas as pl
from jax.experimental.pallas import tpu as pltpu
from jax.experimental.pallas import tpu_sc as plsc

N = 100000
E = 1600000
F_IN = 128
H = 32
C = 2

NC = 2
NS = 16
L = 16

IB = 128
CH = 512
E_PAD = 1605632
NBTOT = E_PAD // IB
DEG_ROWS_W = NBTOT // (NC * NS)
AGG_ROWS_S = NBTOT // NS

_mesh = plsc.VectorSubcoreMesh(core_axis_name="c", subcore_axis_name="s")

NP = 100096
_TILE_FULL = 6400
_TILE_LAST = 4096
_ZROWS = 256


DCH = 4
DNCH = DEG_ROWS_W // DCH


@functools.partial(
    pl.kernel,
    out_type=jax.ShapeDtypeStruct((NC, NP), jnp.float32),
    mesh=_mesh,
    scratch_types=[
        pltpu.VMEM((DCH, IB), jnp.int32),
        pltpu.VMEM((DCH, IB), jnp.int32),
        pltpu.VMEM((DCH * IB,), jnp.float32),
        pltpu.VMEM((DCH * IB,), jnp.float32),
        pltpu.VMEM((_ZROWS,), jnp.float32),
        pltpu.VMEM_SHARED((NP,), jnp.float32),
        pltpu.SemaphoreType.DMA,
        pltpu.SemaphoreType.DMA,
        pltpu.SemaphoreType.DMA,
        pltpu.SemaphoreType.DMA,
    ],
    compiler_params=pltpu.CompilerParams(use_tc_tiling_on_sc=False),
)
def _deg_kernel(dst_hbm, ew_hbm, out_hbm, idx0, idx1, val0, val1, zero_v,
                dacc, seml0, seml1, sems0, sems1):
    cid = lax.axis_index("c")
    sid = lax.axis_index("s")
    wid = sid * NC + cid

    def zfill(i, _):
        zero_v[pl.ds(i * L, L)] = jnp.zeros((L,), jnp.float32)
        return 0
    lax.fori_loop(0, _ZROWS // L, zfill, 0)

    @pl.when(sid < NS - 1)
    def _():
        for r in range(_TILE_FULL // _ZROWS):
            pltpu.sync_copy(zero_v, dacc.at[pl.ds(sid * _TILE_FULL + r * _ZROWS, _ZROWS)])

    @pl.when(sid == NS - 1)
    def _():
        for r in range(_TILE_LAST // _ZROWS):
            pltpu.sync_copy(zero_v, dacc.at[pl.ds((NS - 1) * _TILE_FULL + r * _ZROWS, _ZROWS)])

    plsc.subcore_barrier()

    bufs = ((idx0, val0, seml0, sems0), (idx1, val1, seml1, sems1))

    def lin_start(g, bf):
        row0 = wid * DEG_ROWS_W + g * DCH
        pltpu.async_copy(dst_hbm.at[pl.ds(row0, DCH)], bf[0], bf[2])
        pltpu.async_copy(ew_hbm.at[pl.ds(row0 * IB, DCH * IB)], bf[1], bf[2])

    def lin_wait(bf):
        pltpu.make_async_copy(dst_hbm.at[pl.ds(0, DCH)], bf[0], bf[2]).wait()
        pltpu.make_async_copy(ew_hbm.at[pl.ds(0, DCH * IB)], bf[1], bf[2]).wait()

    lin_start(0, bufs[0])

    def outer(g2, _):
        for b in (0, 1):
            g = g2 * 2 + b
            bf = bufs[b]
            bn = bufs[1 - b]
            lin_wait(bf)

            @pl.when(g < DNCH - 1)
            def _():
                lin_start(g + 1, bn)
            sd = [pltpu.async_copy(bf[1].at[pl.ds(j * IB, IB)],
                                   dacc.at[bf[0].at[j]], bf[3], add=True)
                  for j in range(DCH)]
            for d in sd:
                d.wait()
        return 0
    lax.fori_loop(0, DNCH // 2, outer, 0)

    plsc.subcore_barrier()

    @pl.when(sid < NS - 1)
    def _():
        pltpu.sync_copy(dacc.at[pl.ds(sid * _TILE_FULL, _TILE_FULL)],
                        out_hbm.at[cid, pl.ds(sid * _TILE_FULL, _TILE_FULL)])

    @pl.when(sid == NS - 1)
    def _():
        pltpu.sync_copy(dacc.at[pl.ds((NS - 1) * _TILE_FULL, _TILE_LAST)],
                        out_hbm.at[cid, pl.ds((NS - 1) * _TILE_FULL, _TILE_LAST)])


NB_CH = CH // IB
NCH_T = AGG_ROWS_S // NB_CH


@functools.partial(
    pl.kernel,
    out_type=jax.ShapeDtypeStruct((NP, 128), jnp.float32),
    mesh=_mesh,
    scratch_types=[
        pltpu.VMEM((NB_CH, IB), jnp.int32),
        pltpu.VMEM((NB_CH, IB), jnp.int32),
        pltpu.VMEM((NB_CH, IB), jnp.int32),
        pltpu.VMEM((NB_CH, IB), jnp.int32),
        pltpu.VMEM((CH,), jnp.float32),
        pltpu.VMEM((CH,), jnp.float32),
        pltpu.VMEM((CH, L), jnp.float32),
        pltpu.VMEM((CH, L), jnp.float32),
        pltpu.VMEM((_ZROWS, L), jnp.float32),
        pltpu.VMEM_SHARED((NP, L), jnp.float32),
        pltpu.SemaphoreType.DMA,
        pltpu.SemaphoreType.DMA,
        pltpu.SemaphoreType.DMA,
        pltpu.SemaphoreType.DMA,
        pltpu.SemaphoreType.DMA,
        pltpu.SemaphoreType.DMA,
    ],
    compiler_params=pltpu.CompilerParams(use_tc_tiling_on_sc=False),
)
def _agg_kernel(src_hbm, dst_hbm, ew_hbm, y_hbm, out_hbm,
                sidx0, sidx1, didx0, didx1, ew0, ew1, rows0, rows1,
                zero_v, acc, seml0, seml1, semg0, semg1, sems0, sems1):
    cid = lax.axis_index("c")
    sid = lax.axis_index("s")

    def zfill(i, _):
        zero_v[i] = jnp.zeros((L,), jnp.float32)
        return 0
    lax.fori_loop(0, _ZROWS, zfill, 0)

    @pl.when(sid < NS - 1)
    def _():
        for r in range(_TILE_FULL // _ZROWS):
            pltpu.sync_copy(zero_v, acc.at[pl.ds(sid * _TILE_FULL + r * _ZROWS, _ZROWS)])

    @pl.when(sid == NS - 1)
    def _():
        for r in range(_TILE_LAST // _ZROWS):
            pltpu.sync_copy(zero_v, acc.at[pl.ds((NS - 1) * _TILE_FULL + r * _ZROWS, _ZROWS)])

    plsc.subcore_barrier()

    coff = cid
    bufs = ((sidx0, didx0, ew0, rows0, seml0, semg0, sems0),
            (sidx1, didx1, ew1, rows1, seml1, semg1, sems1))

    def lin_start(g, bf):
        row0 = sid * AGG_ROWS_S + g * NB_CH
        pltpu.async_copy(src_hbm.at[pl.ds(row0, NB_CH)], bf[0], bf[4])
        pltpu.async_copy(dst_hbm.at[pl.ds(row0, NB_CH)], bf[1], bf[4])
        pltpu.async_copy(ew_hbm.at[pl.ds(row0 * IB, CH)], bf[2], bf[4])

    def lin_wait(bf):
        pltpu.make_async_copy(src_hbm.at[pl.ds(0, NB_CH)], bf[0], bf[4]).wait()
        pltpu.make_async_copy(dst_hbm.at[pl.ds(0, NB_CH)], bf[1], bf[4]).wait()
        pltpu.make_async_copy(ew_hbm.at[pl.ds(0, CH)], bf[2], bf[4]).wait()

    def adjust(bf):
        for j in range(NB_CH):
            def adj(k, _):
                v = bf[0][j, pl.ds(k * L, L)]
                t1 = lax.shift_left(lax.shift_right_logical(v, 11), 12)
                t2 = lax.shift_left(v & 511, 3)
                t3 = lax.shift_left(lax.shift_right_logical(v & 2047, 9), 1)
                bf[0][j, pl.ds(k * L, L)] = t1 + t2 + t3 + coff
                return 0
            lax.fori_loop(0, IB // L, adj, 0)

    def gather_start(bf):
        return [pltpu.async_copy(y_hbm.at[bf[0].at[j]],
                                 bf[3].at[pl.ds(j * IB, IB)], bf[5])
                for j in range(NB_CH)]

    def scale_batch(bf, j):
        def scale(gg, _):
            base = j * IB + gg * L
            ewv = bf[2][pl.ds(base, L)]
            for l in range(L):
                bf[3][base + l] = bf[3][base + l] * ewv[l]
            return 0
        lax.fori_loop(0, IB // L, scale, 0)

    def scatter_start(bf, j):
        return pltpu.async_copy(bf[3].at[pl.ds(j * IB, IB)],
                                acc.at[bf[1].at[j]], bf[6], add=True)

    lin_start(0, bufs[0])

    def outer(g2, _):
        for b in (0, 1):
            g = g2 * 2 + b
            bf = bufs[b]
            bn = bufs[1 - b]
            lin_wait(bf)
            adjust(bf)
            gd = gather_start(bf)

            @pl.when(g < NCH_T - 1)
            def _():
                lin_start(g + 1, bn)
            sd = []
            for j in range(NB_CH):
                gd[j].wait()
                scale_batch(bf, j)
                sd.append(scatter_start(bf, j))
            for d in sd:
                d.wait()
        return 0
    lax.fori_loop(0, NCH_T // 2, outer, 0)

    plsc.subcore_barrier()

    @pl.when(sid < NS - 1)
    def _():
        pltpu.sync_copy(
            acc.at[pl.ds(sid * _TILE_FULL, _TILE_FULL)],
            out_hbm.at[pl.ds(sid * _TILE_FULL, _TILE_FULL), pl.ds(cid * L, L)])

    @pl.when(sid == NS - 1)
    def _():
        pltpu.sync_copy(
            acc.at[pl.ds((NS - 1) * _TILE_FULL, _TILE_LAST)],
            out_hbm.at[pl.ds((NS - 1) * _TILE_FULL, _TILE_LAST), pl.ds(cid * L, L)])


BN = 2048
_GRID = (N + BN - 1) // BN


def _dis_from(dp_ref):
    deg = dp_ref[0, :] + dp_ref[1, :] + 1.0
    return lax.rsqrt(deg)


def _pack_y(y):
    q = BN // 4
    return jnp.concatenate([y[0:q], y[q:2 * q], y[2 * q:3 * q], y[3 * q:]],
                           axis=1)


def _pre_body(x_ref, w0_ref, b0_ref, w1_ref, h0_ref, xw1_ref):
    h0 = jnp.maximum(jnp.dot(x_ref[...], w0_ref[...],
                             preferred_element_type=jnp.float32) + b0_ref[...], 0.0)
    h0_ref[...] = h0
    xw1_ref[...] = jnp.dot(h0, w1_ref[...], preferred_element_type=jnp.float32)


def _pack1_body(xw1_ref, dp_ref, y_ref):
    dis = _dis_from(dp_ref)
    y_ref[...] = _pack_y(xw1_ref[...] * dis[:, None])


def _mid_body(acc_ref, xw1_ref, dp_ref, b1_ref, w2_ref, h1_ref, xw2_ref, y2_ref):
    dis = _dis_from(dp_ref)
    acc = acc_ref[...][:, :H]
    xw1 = xw1_ref[...]
    h1 = acc * dis[:, None] + xw1 * (dis * dis)[:, None] + b1_ref[...]
    h1_ref[...] = h1
    xw2 = jnp.dot(h1, w2_ref[...], preferred_element_type=jnp.float32)
    xw2_ref[...] = xw2
    y2_ref[...] = _pack_y(xw2 * dis[:, None])


def _post_body(h0_ref, h1_ref, acc_ref, xw2_ref, dp_ref, b2_ref, wout_ref,
               bout_ref, out_ref):
    dis = _dis_from(dp_ref)
    acc = acc_ref[...][:, :H]
    xw2 = xw2_ref[...]
    h2 = acc * dis[:, None] + xw2 * (dis * dis)[:, None] + b2_ref[...]
    xx = jnp.concatenate([h0_ref[...], h1_ref[...], h2], axis=1)
    logits = jnp.dot(xx, wout_ref[...],
                     preferred_element_type=jnp.float32) + bout_ref[...]
    m = jnp.max(logits, axis=-1, keepdims=True)
    z = logits - m
    out_ref[...] = z - jnp.log(jnp.sum(jnp.exp(z), axis=-1, keepdims=True))


def _row_spec(w):
    return pl.BlockSpec((BN, w), lambda i: (i, 0))


def _full_spec(shape):
    return pl.BlockSpec(shape, lambda i: tuple(0 for _ in shape))


YROWS = BN // 4
NYP = _GRID * YROWS

_dp_spec = pl.BlockSpec((NC, BN), lambda i: (0, i))
_yp_spec = pl.BlockSpec((YROWS, 128), lambda i: (i, 0))
_accp_spec = pl.BlockSpec((BN, 128), lambda i: (i, 0))

_pre = pl.pallas_call(
    _pre_body,
    grid=_GRID,
    in_specs=[_row_spec(F_IN), _full_spec((F_IN, H)), _full_spec((1, H)),
              _full_spec((H, H))],
    out_specs=[_row_spec(H), _row_spec(H)],
    out_shape=[jax.ShapeDtypeStruct((N, H), jnp.float32),
               jax.ShapeDtypeStruct((N, H), jnp.float32)],
)

_pack1 = pl.pallas_call(
    _pack1_body,
    grid=_GRID,
    in_specs=[_row_spec(H), _dp_spec],
    out_specs=_yp_spec,
    out_shape=jax.ShapeDtypeStruct((NYP, 128), jnp.float32),
)

_mid = pl.pallas_call(
    _mid_body,
    grid=_GRID,
    in_specs=[_accp_spec, _row_spec(H), _dp_spec, _full_spec((1, H)),
              _full_spec((H, H))],
    out_specs=[_row_spec(H), _row_spec(H), _yp_spec],
    out_shape=[jax.ShapeDtypeStruct((N, H), jnp.float32),
               jax.ShapeDtypeStruct((N, H), jnp.float32),
               jax.ShapeDtypeStruct((NYP, 128), jnp.float32)],
)

_post = pl.pallas_call(
    _post_body,
    grid=_GRID,
    in_specs=[_row_spec(H), _row_spec(H), _accp_spec, _row_spec(H), _dp_spec,
              _full_spec((1, H)), _full_spec((3 * H, C)), _full_spec((1, C))],
    out_specs=_row_spec(C),
    out_shape=jax.ShapeDtypeStruct((N, C), jnp.float32),
)


def kernel(x, edge_index, edge_weight, W0, b0, W1, b1, W2, b2, Wout, bout):
    src = edge_index[0].astype(jnp.int32)
    dst = edge_index[1].astype(jnp.int32)
    npad = E_PAD - E
    pad_idx = (jnp.arange(npad, dtype=jnp.int32) * 31) % N
    src_p = jnp.concatenate([src, pad_idx]).reshape(NBTOT, IB)
    dst_p = jnp.concatenate([dst, pad_idx]).reshape(NBTOT, IB)
    ew_p = jnp.concatenate(
        [edge_weight.astype(jnp.float32), jnp.zeros((npad,), jnp.float32)])

    dpart = _deg_kernel(dst_p, ew_p)

    h0, xw1 = _pre(x, W0, b0.reshape(1, H), W1)
    y1 = _pack1(xw1, dpart)
    acc1 = _agg_kernel(src_p, dst_p, ew_p, y1.reshape(NYP * 8, L))
    h1, xw2, y2 = _mid(acc1, xw1, dpart, b1.reshape(1, H), W2)
    acc2 = _agg_kernel(src_p, dst_p, ew_p, y2.reshape(NYP * 8, L))
    out = _post(h0, h1, acc2, xw2, dpart, b2.reshape(1, H), Wout,
                bout.reshape(1, C))
    return out


# speedup vs baseline: 36.2704x; 1.0215x over previous
import functools

import jax
import jax.numpy as jnp
from jax import lax
from jax.experimental import pallas as pl
from jax.experimental.pallas import tpu as pltpu
from jax.experimental.pallas import tpu_sc as plsc

N = 100000
E = 1600000
F_IN = 128
H = 32
C = 2

NC = 2
NS = 16
L = 16

IB = 128
CH = 512
E_PAD = 1605632
NBTOT = E_PAD // IB
DEG_ROWS_W = NBTOT // (NC * NS)
AGG_ROWS_S = NBTOT // NS

_mesh = plsc.VectorSubcoreMesh(core_axis_name="c", subcore_axis_name="s")

NP = 100096
_TILE_FULL = 6400
_TILE_LAST = 4096
_ZROWS = 256


EB = 65536
_EGRID = (E_PAD + EB - 1) // EB


def _edges_body(ei_ref, ew_ref, srca_ref, srcb_ref, dst_ref, ewp_ref):
    i = pl.program_id(0)
    idx = i * EB + lax.broadcasted_iota(jnp.int32, (EB,), 0)
    mask = idx < E
    padnode = (idx * 31) % N
    src = jnp.where(mask, ei_ref[0, :], padnode)
    dst_ref[...] = jnp.where(mask, ei_ref[1, :], padnode)
    ewp_ref[...] = jnp.where(mask, ew_ref[...], 0.0)
    f = (lax.shift_left(lax.shift_right_logical(src, 11), 12)
         + lax.shift_left(src & 511, 3)
         + lax.shift_left(lax.shift_right_logical(src & 2047, 9), 1))
    srca_ref[...] = f
    srcb_ref[...] = f + 1


_edges = pl.pallas_call(
    _edges_body,
    grid=_EGRID,
    in_specs=[pl.BlockSpec((2, EB), lambda i: (0, i)),
              pl.BlockSpec((EB,), lambda i: (i,))],
    out_specs=[pl.BlockSpec((EB,), lambda i: (i,))] * 4,
    out_shape=[jax.ShapeDtypeStruct((E_PAD,), jnp.int32),
               jax.ShapeDtypeStruct((E_PAD,), jnp.int32),
               jax.ShapeDtypeStruct((E_PAD,), jnp.int32),
               jax.ShapeDtypeStruct((E_PAD,), jnp.float32)],
)


DCH = 4
DNCH = DEG_ROWS_W // DCH


@functools.partial(
    pl.kernel,
    out_type=jax.ShapeDtypeStruct((NC, NP), jnp.float32),
    mesh=_mesh,
    scratch_types=[
        pltpu.VMEM((DCH, IB), jnp.int32),
        pltpu.VMEM((DCH, IB), jnp.int32),
        pltpu.VMEM((DCH * IB,), jnp.float32),
        pltpu.VMEM((DCH * IB,), jnp.float32),
        pltpu.VMEM((_ZROWS,), jnp.float32),
        pltpu.VMEM_SHARED((NP,), jnp.float32),
        pltpu.SemaphoreType.DMA,
        pltpu.SemaphoreType.DMA,
        pltpu.SemaphoreType.DMA,
        pltpu.SemaphoreType.DMA,
    ],
    compiler_params=pltpu.CompilerParams(use_tc_tiling_on_sc=False),
)
def _deg_kernel(dst_hbm, ew_hbm, out_hbm, idx0, idx1, val0, val1, zero_v,
                dacc, seml0, seml1, sems0, sems1):
    cid = lax.axis_index("c")
    sid = lax.axis_index("s")
    wid = sid * NC + cid

    def zfill(i, _):
        zero_v[pl.ds(i * L, L)] = jnp.zeros((L,), jnp.float32)
        return 0
    lax.fori_loop(0, _ZROWS // L, zfill, 0)

    @pl.when(sid < NS - 1)
    def _():
        for r in range(_TILE_FULL // _ZROWS):
            pltpu.sync_copy(zero_v, dacc.at[pl.ds(sid * _TILE_FULL + r * _ZROWS, _ZROWS)])

    @pl.when(sid == NS - 1)
    def _():
        for r in range(_TILE_LAST // _ZROWS):
            pltpu.sync_copy(zero_v, dacc.at[pl.ds((NS - 1) * _TILE_FULL + r * _ZROWS, _ZROWS)])

    plsc.subcore_barrier()

    bufs = ((idx0, val0, seml0, sems0), (idx1, val1, seml1, sems1))

    def lin_start(g, bf):
        row0 = wid * DEG_ROWS_W + g * DCH
        pltpu.async_copy(dst_hbm.at[pl.ds(row0, DCH)], bf[0], bf[2])
        pltpu.async_copy(ew_hbm.at[pl.ds(row0 * IB, DCH * IB)], bf[1], bf[2])

    def lin_wait(bf):
        pltpu.make_async_copy(dst_hbm.at[pl.ds(0, DCH)], bf[0], bf[2]).wait()
        pltpu.make_async_copy(ew_hbm.at[pl.ds(0, DCH * IB)], bf[1], bf[2]).wait()

    lin_start(0, bufs[0])

    def outer(g2, _):
        for b in (0, 1):
            g = g2 * 2 + b
            bf = bufs[b]
            bn = bufs[1 - b]
            lin_wait(bf)

            @pl.when(g < DNCH - 1)
            def _():
                lin_start(g + 1, bn)
            sd = [pltpu.async_copy(bf[1].at[pl.ds(j * IB, IB)],
                                   dacc.at[bf[0].at[j]], bf[3], add=True)
                  for j in range(DCH)]
            for d in sd:
                d.wait()
        return 0
    lax.fori_loop(0, DNCH // 2, outer, 0)

    plsc.subcore_barrier()

    @pl.when(sid < NS - 1)
    def _():
        pltpu.sync_copy(dacc.at[pl.ds(sid * _TILE_FULL, _TILE_FULL)],
                        out_hbm.at[cid, pl.ds(sid * _TILE_FULL, _TILE_FULL)])

    @pl.when(sid == NS - 1)
    def _():
        pltpu.sync_copy(dacc.at[pl.ds((NS - 1) * _TILE_FULL, _TILE_LAST)],
                        out_hbm.at[cid, pl.ds((NS - 1) * _TILE_FULL, _TILE_LAST)])


NB_CH = CH // IB
NCH_T = AGG_ROWS_S // NB_CH


@functools.partial(
    pl.kernel,
    out_type=jax.ShapeDtypeStruct((NP, 128), jnp.float32),
    mesh=_mesh,
    scratch_types=[
        pltpu.VMEM((NB_CH, IB), jnp.int32),
        pltpu.VMEM((NB_CH, IB), jnp.int32),
        pltpu.VMEM((NB_CH, IB), jnp.int32),
        pltpu.VMEM((NB_CH, IB), jnp.int32),
        pltpu.VMEM((CH,), jnp.float32),
        pltpu.VMEM((CH,), jnp.float32),
        pltpu.VMEM((CH, L), jnp.float32),
        pltpu.VMEM((CH, L), jnp.float32),
        pltpu.VMEM((_ZROWS, L), jnp.float32),
        pltpu.VMEM_SHARED((NP, L), jnp.float32),
        pltpu.SemaphoreType.DMA,
        pltpu.SemaphoreType.DMA,
        pltpu.SemaphoreType.DMA,
        pltpu.SemaphoreType.DMA,
        pltpu.SemaphoreType.DMA,
        pltpu.SemaphoreType.DMA,
    ],
    compiler_params=pltpu.CompilerParams(use_tc_tiling_on_sc=False),
)
def _agg_kernel(srca_hbm, srcb_hbm, dst_hbm, ew_hbm, y_hbm, out_hbm,
                sidx0, sidx1, didx0, didx1, ew0, ew1, rows0, rows1,
                zero_v, acc, seml0, seml1, semg0, semg1, sems0, sems1):
    cid = lax.axis_index("c")
    sid = lax.axis_index("s")

    def zfill(i, _):
        zero_v[i] = jnp.zeros((L,), jnp.float32)
        return 0
    lax.fori_loop(0, _ZROWS, zfill, 0)

    @pl.when(sid < NS - 1)
    def _():
        for r in range(_TILE_FULL // _ZROWS):
            pltpu.sync_copy(zero_v, acc.at[pl.ds(sid * _TILE_FULL + r * _ZROWS, _ZROWS)])

    @pl.when(sid == NS - 1)
    def _():
        for r in range(_TILE_LAST // _ZROWS):
            pltpu.sync_copy(zero_v, acc.at[pl.ds((NS - 1) * _TILE_FULL + r * _ZROWS, _ZROWS)])

    plsc.subcore_barrier()

    bufs = ((sidx0, didx0, ew0, rows0, seml0, semg0, sems0),
            (sidx1, didx1, ew1, rows1, seml1, semg1, sems1))

    def lin_start(g, bf):
        row0 = sid * AGG_ROWS_S + g * NB_CH

        @pl.when(cid == 0)
        def _():
            pltpu.async_copy(srca_hbm.at[pl.ds(row0, NB_CH)], bf[0], bf[4])

        @pl.when(cid == 1)
        def _():
            pltpu.async_copy(srcb_hbm.at[pl.ds(row0, NB_CH)], bf[0], bf[4])
        pltpu.async_copy(dst_hbm.at[pl.ds(row0, NB_CH)], bf[1], bf[4])
        pltpu.async_copy(ew_hbm.at[pl.ds(row0 * IB, CH)], bf[2], bf[4])

    def lin_wait(bf):
        pltpu.make_async_copy(srca_hbm.at[pl.ds(0, NB_CH)], bf[0], bf[4]).wait()
        pltpu.make_async_copy(dst_hbm.at[pl.ds(0, NB_CH)], bf[1], bf[4]).wait()
        pltpu.make_async_copy(ew_hbm.at[pl.ds(0, CH)], bf[2], bf[4]).wait()

    def gather_start(bf):
        return [pltpu.async_copy(y_hbm.at[bf[0].at[j]],
                                 bf[3].at[pl.ds(j * IB, IB)], bf[5])
                for j in range(NB_CH)]

    def scale_batch(bf, j):
        def scale(gg, _):
            base = j * IB + gg * L
            ewv = bf[2][pl.ds(base, L)]
            for l in range(L):
                bf[3][base + l] = bf[3][base + l] * ewv[l]
            return 0
        lax.fori_loop(0, IB // L, scale, 0)

    def scatter_start(bf, j):
        return pltpu.async_copy(bf[3].at[pl.ds(j * IB, IB)],
                                acc.at[bf[1].at[j]], bf[6], add=True)

    lin_start(0, bufs[0])

    def outer(g2, _):
        for b in (0, 1):
            g = g2 * 2 + b
            bf = bufs[b]
            bn = bufs[1 - b]
            lin_wait(bf)
            gd = gather_start(bf)

            @pl.when(g < NCH_T - 1)
            def _():
                lin_start(g + 1, bn)
            sd = []
            for j in range(NB_CH):
                gd[j].wait()
                scale_batch(bf, j)
                sd.append(scatter_start(bf, j))
            for d in sd:
                d.wait()
        return 0
    lax.fori_loop(0, NCH_T // 2, outer, 0)

    plsc.subcore_barrier()

    @pl.when(sid < NS - 1)
    def _():
        pltpu.sync_copy(
            acc.at[pl.ds(sid * _TILE_FULL, _TILE_FULL)],
            out_hbm.at[pl.ds(sid * _TILE_FULL, _TILE_FULL), pl.ds(cid * L, L)])

    @pl.when(sid == NS - 1)
    def _():
        pltpu.sync_copy(
            acc.at[pl.ds((NS - 1) * _TILE_FULL, _TILE_LAST)],
            out_hbm.at[pl.ds((NS - 1) * _TILE_FULL, _TILE_LAST), pl.ds(cid * L, L)])


BN = 2048
_GRID = (N + BN - 1) // BN


def _dis_from(dp_ref):
    deg = dp_ref[0, :] + dp_ref[1, :] + 1.0
    return lax.rsqrt(deg)


def _pack_y(y):
    q = BN // 4
    return jnp.concatenate([y[0:q], y[q:2 * q], y[2 * q:3 * q], y[3 * q:]],
                           axis=1)


def _pre_body(x_ref, w0_ref, b0_ref, w1_ref, h0_ref, xw1_ref):
    h0 = jnp.maximum(jnp.dot(x_ref[...], w0_ref[...],
                             preferred_element_type=jnp.float32) + b0_ref[...], 0.0)
    h0_ref[...] = h0
    xw1_ref[...] = jnp.dot(h0, w1_ref[...], preferred_element_type=jnp.float32)


def _pack1_body(xw1_ref, dp_ref, y_ref):
    dis = _dis_from(dp_ref)
    y_ref[...] = _pack_y(xw1_ref[...] * dis[:, None])


def _mid_body(acc_ref, xw1_ref, dp_ref, b1_ref, w2_ref, h1_ref, xw2_ref, y2_ref):
    dis = _dis_from(dp_ref)
    acc = acc_ref[...][:, :H]
    xw1 = xw1_ref[...]
    h1 = acc * dis[:, None] + xw1 * (dis * dis)[:, None] + b1_ref[...]
    h1_ref[...] = h1
    xw2 = jnp.dot(h1, w2_ref[...], preferred_element_type=jnp.float32)
    xw2_ref[...] = xw2
    y2_ref[...] = _pack_y(xw2 * dis[:, None])


def _post_body(h0_ref, h1_ref, acc_ref, xw2_ref, dp_ref, b2_ref, wout_ref,
               bout_ref, out_ref):
    dis = _dis_from(dp_ref)
    acc = acc_ref[...][:, :H]
    xw2 = xw2_ref[...]
    h2 = acc * dis[:, None] + xw2 * (dis * dis)[:, None] + b2_ref[...]
    xx = jnp.concatenate([h0_ref[...], h1_ref[...], h2], axis=1)
    logits = jnp.dot(xx, wout_ref[...],
                     preferred_element_type=jnp.float32) + bout_ref[...]
    m = jnp.max(logits, axis=-1, keepdims=True)
    z = logits - m
    out_ref[...] = z - jnp.log(jnp.sum(jnp.exp(z), axis=-1, keepdims=True))


def _row_spec(w):
    return pl.BlockSpec((BN, w), lambda i: (i, 0))


def _full_spec(shape):
    return pl.BlockSpec(shape, lambda i: tuple(0 for _ in shape))


YROWS = BN // 4
NYP = _GRID * YROWS

_dp_spec = pl.BlockSpec((NC, BN), lambda i: (0, i))
_yp_spec = pl.BlockSpec((YROWS, 128), lambda i: (i, 0))
_accp_spec = pl.BlockSpec((BN, 128), lambda i: (i, 0))

_pre = pl.pallas_call(
    _pre_body,
    grid=_GRID,
    in_specs=[_row_spec(F_IN), _full_spec((F_IN, H)), _full_spec((1, H)),
              _full_spec((H, H))],
    out_specs=[_row_spec(H), _row_spec(H)],
    out_shape=[jax.ShapeDtypeStruct((N, H), jnp.float32),
               jax.ShapeDtypeStruct((N, H), jnp.float32)],
)

_pack1 = pl.pallas_call(
    _pack1_body,
    grid=_GRID,
    in_specs=[_row_spec(H), _dp_spec],
    out_specs=_yp_spec,
    out_shape=jax.ShapeDtypeStruct((NYP, 128), jnp.float32),
)

_mid = pl.pallas_call(
    _mid_body,
    grid=_GRID,
    in_specs=[_accp_spec, _row_spec(H), _dp_spec, _full_spec((1, H)),
              _full_spec((H, H))],
    out_specs=[_row_spec(H), _row_spec(H), _yp_spec],
    out_shape=[jax.ShapeDtypeStruct((N, H), jnp.float32),
               jax.ShapeDtypeStruct((N, H), jnp.float32),
               jax.ShapeDtypeStruct((NYP, 128), jnp.float32)],
)

_post = pl.pallas_call(
    _post_body,
    grid=_GRID,
    in_specs=[_row_spec(H), _row_spec(H), _accp_spec, _row_spec(H), _dp_spec,
              _full_spec((1, H)), _full_spec((3 * H, C)), _full_spec((1, C))],
    out_specs=_row_spec(C),
    out_shape=jax.ShapeDtypeStruct((N, C), jnp.float32),
)


def kernel(x, edge_index, edge_weight, W0, b0, W1, b1, W2, b2, Wout, bout):
    srca, srcb, dst_e, ew_e = _edges(edge_index.astype(jnp.int32),
                                     edge_weight.astype(jnp.float32))
    srca_p = srca.reshape(NBTOT, IB)
    srcb_p = srcb.reshape(NBTOT, IB)
    dst_p = dst_e.reshape(NBTOT, IB)
    ew_p = ew_e

    dpart = _deg_kernel(dst_p, ew_p)

    h0, xw1 = _pre(x, W0, b0.reshape(1, H), W1)
    y1 = _pack1(xw1, dpart)
    acc1 = _agg_kernel(srca_p, srcb_p, dst_p, ew_p, y1.reshape(NYP * 8, L))
    h1, xw2, y2 = _mid(acc1, xw1, dpart, b1.reshape(1, H), W2)
    acc2 = _agg_kernel(srca_p, srcb_p, dst_p, ew_p, y2.reshape(NYP * 8, L))
    out = _post(h0, h1, acc2, xw2, dpart, b2.reshape(1, H), Wout,
                bout.reshape(1, C))
    return out


# speedup vs baseline: 37.9134x; 1.0453x over previous
import functools

import jax
import jax.numpy as jnp
from jax import lax
from jax.experimental import pallas as pl
from jax.experimental.pallas import tpu as pltpu
from jax.experimental.pallas import tpu_sc as plsc

N = 100000
E = 1600000
F_IN = 128
H = 32
C = 2

NC = 2
NS = 16
L = 16

IB = 128
CH = 512
E_PAD = 1605632
NBTOT = E_PAD // IB
DEG_ROWS_W = NBTOT // (NC * NS)
AGG_ROWS_S = NBTOT // NS

_mesh = plsc.VectorSubcoreMesh(core_axis_name="c", subcore_axis_name="s")

NP = 100096
_TILE_FULL = 6400
_TILE_LAST = 4096
_ZROWS = 256


EB = 65536
_EGRID = (E_PAD + EB - 1) // EB


def _edges_body(ei_ref, ew_ref, srca_ref, srcb_ref, dst_ref, ewp_ref):
    i = pl.program_id(0)
    idx = i * EB + lax.broadcasted_iota(jnp.int32, (EB,), 0)
    mask = idx < E
    padnode = (idx * 31) % N
    src = jnp.where(mask, ei_ref[0, :], padnode)
    dst_ref[...] = jnp.where(mask, ei_ref[1, :], padnode)
    ewp_ref[...] = jnp.where(mask, ew_ref[...], 0.0)
    f = (lax.shift_left(lax.shift_right_logical(src, 11), 12)
         + lax.shift_left(src & 511, 3)
         + lax.shift_left(lax.shift_right_logical(src & 2047, 9), 1))
    srca_ref[...] = f
    srcb_ref[...] = f + 1


_edges = pl.pallas_call(
    _edges_body,
    grid=_EGRID,
    in_specs=[pl.BlockSpec((2, EB), lambda i: (0, i)),
              pl.BlockSpec((EB,), lambda i: (i,))],
    out_specs=[pl.BlockSpec((EB,), lambda i: (i,))] * 4,
    out_shape=[jax.ShapeDtypeStruct((E_PAD,), jnp.int32),
               jax.ShapeDtypeStruct((E_PAD,), jnp.int32),
               jax.ShapeDtypeStruct((E_PAD,), jnp.int32),
               jax.ShapeDtypeStruct((E_PAD,), jnp.float32)],
)


DCH = 4
DNCH = DEG_ROWS_W // DCH


@functools.partial(
    pl.kernel,
    out_type=jax.ShapeDtypeStruct((NC, NP), jnp.float32),
    mesh=_mesh,
    scratch_types=[
        pltpu.VMEM((DCH, IB), jnp.int32),
        pltpu.VMEM((DCH, IB), jnp.int32),
        pltpu.VMEM((DCH * IB,), jnp.float32),
        pltpu.VMEM((DCH * IB,), jnp.float32),
        pltpu.VMEM((_ZROWS,), jnp.float32),
        pltpu.VMEM_SHARED((NP,), jnp.float32),
        pltpu.SemaphoreType.DMA,
        pltpu.SemaphoreType.DMA,
        pltpu.SemaphoreType.DMA,
        pltpu.SemaphoreType.DMA,
    ],
    compiler_params=pltpu.CompilerParams(use_tc_tiling_on_sc=False),
)
def _deg_kernel(dst_hbm, ew_hbm, out_hbm, idx0, idx1, val0, val1, zero_v,
                dacc, seml0, seml1, sems0, sems1):
    cid = lax.axis_index("c")
    sid = lax.axis_index("s")
    wid = sid * NC + cid

    def zfill(i, _):
        zero_v[pl.ds(i * L, L)] = jnp.zeros((L,), jnp.float32)
        return 0
    lax.fori_loop(0, _ZROWS // L, zfill, 0)

    @pl.when(sid < NS - 1)
    def _():
        for r in range(_TILE_FULL // _ZROWS):
            pltpu.sync_copy(zero_v, dacc.at[pl.ds(sid * _TILE_FULL + r * _ZROWS, _ZROWS)])

    @pl.when(sid == NS - 1)
    def _():
        for r in range(_TILE_LAST // _ZROWS):
            pltpu.sync_copy(zero_v, dacc.at[pl.ds((NS - 1) * _TILE_FULL + r * _ZROWS, _ZROWS)])

    plsc.subcore_barrier()

    bufs = ((idx0, val0, seml0, sems0), (idx1, val1, seml1, sems1))

    def lin_start(g, bf):
        row0 = wid * DEG_ROWS_W + g * DCH
        pltpu.async_copy(dst_hbm.at[pl.ds(row0, DCH)], bf[0], bf[2])
        pltpu.async_copy(ew_hbm.at[pl.ds(row0 * IB, DCH * IB)], bf[1], bf[2])

    def lin_wait(bf):
        pltpu.make_async_copy(dst_hbm.at[pl.ds(0, DCH)], bf[0], bf[2]).wait()
        pltpu.make_async_copy(ew_hbm.at[pl.ds(0, DCH * IB)], bf[1], bf[2]).wait()

    lin_start(0, bufs[0])

    def outer(g2, _):
        for b in (0, 1):
            g = g2 * 2 + b
            bf = bufs[b]
            bn = bufs[1 - b]
            lin_wait(bf)

            @pl.when(g < DNCH - 1)
            def _():
                lin_start(g + 1, bn)
            sd = [pltpu.async_copy(bf[1].at[pl.ds(j * IB, IB)],
                                   dacc.at[bf[0].at[j]], bf[3], add=True)
                  for j in range(DCH)]
            for d in sd:
                d.wait()
        return 0
    lax.fori_loop(0, DNCH // 2, outer, 0)

    plsc.subcore_barrier()

    @pl.when(sid < NS - 1)
    def _():
        pltpu.sync_copy(dacc.at[pl.ds(sid * _TILE_FULL, _TILE_FULL)],
                        out_hbm.at[cid, pl.ds(sid * _TILE_FULL, _TILE_FULL)])

    @pl.when(sid == NS - 1)
    def _():
        pltpu.sync_copy(dacc.at[pl.ds((NS - 1) * _TILE_FULL, _TILE_LAST)],
                        out_hbm.at[cid, pl.ds((NS - 1) * _TILE_FULL, _TILE_LAST)])


NB_CH = CH // IB
NCH_T = AGG_ROWS_S // NB_CH


@functools.partial(
    pl.kernel,
    out_type=jax.ShapeDtypeStruct((NP, 128), jnp.float32),
    mesh=_mesh,
    scratch_types=[
        pltpu.VMEM((NB_CH, IB), jnp.int32),
        pltpu.VMEM((NB_CH, IB), jnp.int32),
        pltpu.VMEM((NB_CH, IB), jnp.int32),
        pltpu.VMEM((NB_CH, IB), jnp.int32),
        pltpu.VMEM((CH,), jnp.float32),
        pltpu.VMEM((CH,), jnp.float32),
        pltpu.VMEM((CH, L), jnp.float32),
        pltpu.VMEM((CH, L), jnp.float32),
        pltpu.VMEM((_ZROWS, L), jnp.float32),
        pltpu.VMEM_SHARED((NP, L), jnp.float32),
        pltpu.SemaphoreType.DMA,
        pltpu.SemaphoreType.DMA,
        pltpu.SemaphoreType.DMA,
        pltpu.SemaphoreType.DMA,
        pltpu.SemaphoreType.DMA,
        pltpu.SemaphoreType.DMA,
    ],
    compiler_params=pltpu.CompilerParams(use_tc_tiling_on_sc=False),
)
def _agg_kernel(srca_hbm, srcb_hbm, dst_hbm, ew_hbm, y_hbm, out_hbm,
                sidx0, sidx1, didx0, didx1, ew0, ew1, rows0, rows1,
                zero_v, acc, seml0, seml1, semg0, semg1, sems0, sems1):
    cid = lax.axis_index("c")
    sid = lax.axis_index("s")

    def zfill(i, _):
        zero_v[i] = jnp.zeros((L,), jnp.float32)
        return 0
    lax.fori_loop(0, _ZROWS, zfill, 0)

    @pl.when(sid < NS - 1)
    def _():
        for r in range(_TILE_FULL // _ZROWS):
            pltpu.sync_copy(zero_v, acc.at[pl.ds(sid * _TILE_FULL + r * _ZROWS, _ZROWS)])

    @pl.when(sid == NS - 1)
    def _():
        for r in range(_TILE_LAST // _ZROWS):
            pltpu.sync_copy(zero_v, acc.at[pl.ds((NS - 1) * _TILE_FULL + r * _ZROWS, _ZROWS)])

    plsc.subcore_barrier()

    bufs = ((sidx0, didx0, ew0, rows0, seml0, semg0, sems0),
            (sidx1, didx1, ew1, rows1, seml1, semg1, sems1))

    def lin_start(g, bf):
        row0 = sid * AGG_ROWS_S + g * NB_CH

        @pl.when(cid == 0)
        def _():
            pltpu.async_copy(srca_hbm.at[pl.ds(row0, NB_CH)], bf[0], bf[4])

        @pl.when(cid == 1)
        def _():
            pltpu.async_copy(srcb_hbm.at[pl.ds(row0, NB_CH)], bf[0], bf[4])
        pltpu.async_copy(dst_hbm.at[pl.ds(row0, NB_CH)], bf[1], bf[4])
        pltpu.async_copy(ew_hbm.at[pl.ds(row0 * IB, CH)], bf[2], bf[4])

    def lin_wait(bf):
        pltpu.make_async_copy(srca_hbm.at[pl.ds(0, NB_CH)], bf[0], bf[4]).wait()
        pltpu.make_async_copy(dst_hbm.at[pl.ds(0, NB_CH)], bf[1], bf[4]).wait()
        pltpu.make_async_copy(ew_hbm.at[pl.ds(0, CH)], bf[2], bf[4]).wait()

    def gather_start(bf):
        for j in range(NB_CH):
            pltpu.async_copy(y_hbm.at[bf[0].at[j]],
                             bf[3].at[pl.ds(j * IB, IB)], bf[5])

    def gather_wait1(bf, j):
        pltpu.make_async_copy(y_hbm.at[bf[0].at[j]],
                              bf[3].at[pl.ds(j * IB, IB)], bf[5]).wait()

    def scale_batch(bf, j):
        def scale(gg, _):
            base = j * IB + gg * L
            ewv = bf[2][pl.ds(base, L)]
            for l in range(L):
                bf[3][base + l] = bf[3][base + l] * ewv[l]
            return 0
        lax.fori_loop(0, IB // L, scale, 0)

    def scatter_start(bf, j):
        return pltpu.async_copy(bf[3].at[pl.ds(j * IB, IB)],
                                acc.at[bf[1].at[j]], bf[6], add=True)

    lin_start(0, bufs[0])
    lin_wait(bufs[0])
    gather_start(bufs[0])

    def outer(g2, _):
        for b in (0, 1):
            g = g2 * 2 + b
            bf = bufs[b]
            bn = bufs[1 - b]

            @pl.when(g < NCH_T - 1)
            def _():
                lin_start(g + 1, bn)
            sd = []
            for j in range(NB_CH):
                gather_wait1(bf, j)
                scale_batch(bf, j)
                sd.append(scatter_start(bf, j))

            @pl.when(g < NCH_T - 1)
            def _():
                lin_wait(bn)
                gather_start(bn)
            for d in sd:
                d.wait()
        return 0
    lax.fori_loop(0, NCH_T // 2, outer, 0)

    plsc.subcore_barrier()

    @pl.when(sid < NS - 1)
    def _():
        pltpu.sync_copy(
            acc.at[pl.ds(sid * _TILE_FULL, _TILE_FULL)],
            out_hbm.at[pl.ds(sid * _TILE_FULL, _TILE_FULL), pl.ds(cid * L, L)])

    @pl.when(sid == NS - 1)
    def _():
        pltpu.sync_copy(
            acc.at[pl.ds((NS - 1) * _TILE_FULL, _TILE_LAST)],
            out_hbm.at[pl.ds((NS - 1) * _TILE_FULL, _TILE_LAST), pl.ds(cid * L, L)])


BN = 2048
_GRID = (N + BN - 1) // BN


def _dis_from(dp_ref):
    deg = dp_ref[0, :] + dp_ref[1, :] + 1.0
    return lax.rsqrt(deg)


def _pack_y(y):
    q = BN // 4
    return jnp.concatenate([y[0:q], y[q:2 * q], y[2 * q:3 * q], y[3 * q:]],
                           axis=1)


def _pre_body(x_ref, w0_ref, b0_ref, w1_ref, h0_ref, xw1_ref):
    h0 = jnp.maximum(jnp.dot(x_ref[...], w0_ref[...],
                             preferred_element_type=jnp.float32) + b0_ref[...], 0.0)
    h0_ref[...] = h0
    xw1_ref[...] = jnp.dot(h0, w1_ref[...], preferred_element_type=jnp.float32)


def _pack1_body(xw1_ref, dp_ref, y_ref):
    dis = _dis_from(dp_ref)
    y_ref[...] = _pack_y(xw1_ref[...] * dis[:, None])


def _mid_body(acc_ref, xw1_ref, dp_ref, b1_ref, w2_ref, h1_ref, xw2_ref, y2_ref):
    dis = _dis_from(dp_ref)
    acc = acc_ref[...][:, :H]
    xw1 = xw1_ref[...]
    h1 = acc * dis[:, None] + xw1 * (dis * dis)[:, None] + b1_ref[...]
    h1_ref[...] = h1
    xw2 = jnp.dot(h1, w2_ref[...], preferred_element_type=jnp.float32)
    xw2_ref[...] = xw2
    y2_ref[...] = _pack_y(xw2 * dis[:, None])


def _post_body(h0_ref, h1_ref, acc_ref, xw2_ref, dp_ref, b2_ref, wout_ref,
               bout_ref, out_ref):
    dis = _dis_from(dp_ref)
    acc = acc_ref[...][:, :H]
    xw2 = xw2_ref[...]
    h2 = acc * dis[:, None] + xw2 * (dis * dis)[:, None] + b2_ref[...]
    xx = jnp.concatenate([h0_ref[...], h1_ref[...], h2], axis=1)
    logits = jnp.dot(xx, wout_ref[...],
                     preferred_element_type=jnp.float32) + bout_ref[...]
    m = jnp.max(logits, axis=-1, keepdims=True)
    z = logits - m
    out_ref[...] = z - jnp.log(jnp.sum(jnp.exp(z), axis=-1, keepdims=True))


def _row_spec(w):
    return pl.BlockSpec((BN, w), lambda i: (i, 0))


def _full_spec(shape):
    return pl.BlockSpec(shape, lambda i: tuple(0 for _ in shape))


YROWS = BN // 4
NYP = _GRID * YROWS

_dp_spec = pl.BlockSpec((NC, BN), lambda i: (0, i))
_yp_spec = pl.BlockSpec((YROWS, 128), lambda i: (i, 0))
_accp_spec = pl.BlockSpec((BN, 128), lambda i: (i, 0))

_pre = pl.pallas_call(
    _pre_body,
    grid=_GRID,
    in_specs=[_row_spec(F_IN), _full_spec((F_IN, H)), _full_spec((1, H)),
              _full_spec((H, H))],
    out_specs=[_row_spec(H), _row_spec(H)],
    out_shape=[jax.ShapeDtypeStruct((N, H), jnp.float32),
               jax.ShapeDtypeStruct((N, H), jnp.float32)],
)

_pack1 = pl.pallas_call(
    _pack1_body,
    grid=_GRID,
    in_specs=[_row_spec(H), _dp_spec],
    out_specs=_yp_spec,
    out_shape=jax.ShapeDtypeStruct((NYP, 128), jnp.float32),
)

_mid = pl.pallas_call(
    _mid_body,
    grid=_GRID,
    in_specs=[_accp_spec, _row_spec(H), _dp_spec, _full_spec((1, H)),
              _full_spec((H, H))],
    out_specs=[_row_spec(H), _row_spec(H), _yp_spec],
    out_shape=[jax.ShapeDtypeStruct((N, H), jnp.float32),
               jax.ShapeDtypeStruct((N, H), jnp.float32),
               jax.ShapeDtypeStruct((NYP, 128), jnp.float32)],
)

_post = pl.pallas_call(
    _post_body,
    grid=_GRID,
    in_specs=[_row_spec(H), _row_spec(H), _accp_spec, _row_spec(H), _dp_spec,
              _full_spec((1, H)), _full_spec((3 * H, C)), _full_spec((1, C))],
    out_specs=_row_spec(C),
    out_shape=jax.ShapeDtypeStruct((N, C), jnp.float32),
)


def kernel(x, edge_index, edge_weight, W0, b0, W1, b1, W2, b2, Wout, bout):
    srca, srcb, dst_e, ew_e = _edges(edge_index.astype(jnp.int32),
                                     edge_weight.astype(jnp.float32))
    srca_p = srca.reshape(NBTOT, IB)
    srcb_p = srcb.reshape(NBTOT, IB)
    dst_p = dst_e.reshape(NBTOT, IB)
    ew_p = ew_e

    dpart = _deg_kernel(dst_p, ew_p)

    h0, xw1 = _pre(x, W0, b0.reshape(1, H), W1)
    y1 = _pack1(xw1, dpart)
    acc1 = _agg_kernel(srca_p, srcb_p, dst_p, ew_p, y1.reshape(NYP * 8, L))
    h1, xw2, y2 = _mid(acc1, xw1, dpart, b1.reshape(1, H), W2)
    acc2 = _agg_kernel(srca_p, srcb_p, dst_p, ew_p, y2.reshape(NYP * 8, L))
    out = _post(h0, h1, acc2, xw2, dpart, b2.reshape(1, H), Wout,
                bout.reshape(1, C))
    return out
